# Initial kernel scaffold; baseline (speedup 1.0000x reference)
import jax
import jax.numpy as jnp
from jax.experimental import pallas as pl


def kernel(mu, Wq, bq, Wk, bk, Wv, bv, Wm1, bm1, Wm2, bm2, Wo, bo):
    raise NotImplementedError("write your pallas kernel here")


# speedup vs baseline: 5.9615x; 5.9615x over previous
import math

import jax
import jax.numpy as jnp
from jax import lax
from jax.experimental import pallas as pl
from jax.experimental.pallas import tpu as pltpu

_WINDOW = 4
_K = 8
_QBLK = 256


def _fold_kernel(wv_ref, wm2_ref, wo_ref, bv_ref, bm2_ref, bo_ref,
                 wvpT_ref, wlT_ref, bvp_ref, ball_ref):
    D = wv_ref.shape[0]
    wo = wo_ref[...]
    wol = wo[:, :D]
    wog = wo[:, D:]
    wv = wv_ref[...]
    wm2 = wm2_ref[...]
    wvpT_ref[...] = lax.dot_general(
        wv, wog, (((0,), (1,)), ((), ())), preferred_element_type=jnp.float32)
    wlT_ref[...] = lax.dot_general(
        wm2, wol, (((0,), (1,)), ((), ())), preferred_element_type=jnp.float32)
    bvp_ref[...] = lax.dot_general(
        bv_ref[...], wog, (((1,), (1,)), ((), ())),
        preferred_element_type=jnp.float32)
    ball_ref[...] = bo_ref[...] + lax.dot_general(
        bm2_ref[...], wol, (((1,), (1,)), ((), ())),
        preferred_element_type=jnp.float32)


def _proj_kernel(x_ref, w_ref, b_ref, o_ref):
    o_ref[...] = jnp.dot(
        x_ref[...], w_ref[...], preferred_element_type=jnp.float32
    ) + b_ref[...]


def _attn_kernel(q_ref, k_ref, v_ref, g_ref):
    qi = pl.program_id(1)
    QB = q_ref.shape[1]
    N = k_ref.shape[1]
    D = q_ref.shape[2]
    scale = 1.0 / math.sqrt(D)
    q = q_ref[0]
    k = k_ref[0]
    s = lax.dot_general(
        q, k, (((1,), (1,)), ((), ())), preferred_element_type=jnp.float32
    ) * scale
    row = qi * QB + lax.broadcasted_iota(jnp.int32, (QB, N), 0)
    col = lax.broadcasted_iota(jnp.int32, (QB, N), 1)
    neg = jnp.float32(-jnp.inf)
    s = jnp.where(col <= row, s, neg)
    work = s
    m0 = None
    for j in range(_K):
        m = jnp.max(work, axis=1, keepdims=True)
        if j == 0:
            m0 = m
        am = jnp.min(jnp.where(work == m, col, N), axis=1, keepdims=True)
        work = jnp.where(col == am, neg, work)
    mask = work != s
    p = jnp.where(mask, jnp.exp(s - m0), 0.0)
    z = jnp.sum(p, axis=1, keepdims=True)
    g = lax.dot_general(
        p, v_ref[0], (((1,), (0,)), ((), ())),
        preferred_element_type=jnp.float32)
    g_ref[0] = g / z


def _local_kernel(a_ref, bc_ref, bp_ref, g_ref, wlT_ref, bm1_ref, ball_ref,
                  o_ref):
    qi = pl.program_id(1)
    QB = a_ref.shape[1]
    a = a_ref[0] + bm1_ref[...]
    cur = bc_ref[0]
    prev = jnp.where(qi == 0, 0.0, bp_ref[0])
    acc = jnp.zeros_like(a)
    for w in range(1, _WINDOW + 1):
        sh = jnp.concatenate([prev[QB - w:, :], cur[:QB - w, :]], axis=0)
        acc = acc + jax.nn.silu(a + sh)
    pre = acc * (1.0 / _WINDOW)
    o_ref[0] = jnp.dot(
        pre, wlT_ref[...], preferred_element_type=jnp.float32
    ) + ball_ref[...] + g_ref[0]


def kernel(mu, Wq, bq, Wk, bk, Wv, bv, Wm1, bm1, Wm2, bm2, Wo, bo):
    B, N, D = mu.shape
    f32 = jnp.float32

    wvpT, wlT, bvp, ball = pl.pallas_call(
        _fold_kernel,
        out_shape=[
            jax.ShapeDtypeStruct((D, D), f32),
            jax.ShapeDtypeStruct((D, D), f32),
            jax.ShapeDtypeStruct((1, D), f32),
            jax.ShapeDtypeStruct((1, D), f32),
        ],
    )(Wv, Wm2, Wo, bv[None, :], bm2[None, :], bo[None, :])

    wcat = jnp.concatenate(
        [Wq.T, Wk.T, wvpT, Wm1[:, :D].T, Wm1[:, D:].T], axis=1)
    bcat = jnp.concatenate(
        [bq[None, :], bk[None, :], bvp,
         jnp.zeros((1, 2 * D), f32)], axis=1)
    mu_flat = mu.reshape(B * N, D)
    rblk = _QBLK
    proj = pl.pallas_call(
        _proj_kernel,
        grid=(B * N // rblk,),
        in_specs=[
            pl.BlockSpec((rblk, D), lambda i: (i, 0)),
            pl.BlockSpec((D, 5 * D), lambda i: (0, 0)),
            pl.BlockSpec((1, 5 * D), lambda i: (0, 0)),
        ],
        out_specs=pl.BlockSpec((rblk, 5 * D), lambda i: (i, 0)),
        out_shape=jax.ShapeDtypeStruct((B * N, 5 * D), f32),
    )(mu_flat, wcat, bcat)

    q = proj[:, 0 * D:1 * D].reshape(B, N, D)
    kk = proj[:, 1 * D:2 * D].reshape(B, N, D)
    vp = proj[:, 2 * D:3 * D].reshape(B, N, D)
    aa = proj[:, 3 * D:4 * D].reshape(B, N, D)
    bn = proj[:, 4 * D:5 * D].reshape(B, N, D)

    nblk = N // _QBLK
    g = pl.pallas_call(
        _attn_kernel,
        grid=(B, nblk),
        in_specs=[
            pl.BlockSpec((1, _QBLK, D), lambda b, i: (b, i, 0)),
            pl.BlockSpec((1, N, D), lambda b, i: (b, 0, 0)),
            pl.BlockSpec((1, N, D), lambda b, i: (b, 0, 0)),
        ],
        out_specs=pl.BlockSpec((1, _QBLK, D), lambda b, i: (b, i, 0)),
        out_shape=jax.ShapeDtypeStruct((B, N, D), f32),
    )(q, kk, vp)

    out = pl.pallas_call(
        _local_kernel,
        grid=(B, nblk),
        in_specs=[
            pl.BlockSpec((1, _QBLK, D), lambda b, i: (b, i, 0)),
            pl.BlockSpec((1, _QBLK, D), lambda b, i: (b, i, 0)),
            pl.BlockSpec((1, _QBLK, D), lambda b, i: (b, jnp.maximum(i - 1, 0), 0)),
            pl.BlockSpec((1, _QBLK, D), lambda b, i: (b, i, 0)),
            pl.BlockSpec((D, D), lambda b, i: (0, 0)),
            pl.BlockSpec((1, D), lambda b, i: (0, 0)),
            pl.BlockSpec((1, D), lambda b, i: (0, 0)),
        ],
        out_specs=pl.BlockSpec((1, _QBLK, D), lambda b, i: (b, i, 0)),
        out_shape=jax.ShapeDtypeStruct((B, N, D), f32),
    )(aa, bn, bn, g, wlT, bm1[None, :], ball)
    return out


# speedup vs baseline: 9.1683x; 1.5379x over previous
import math

import jax
import jax.numpy as jnp
from jax import lax
from jax.experimental import pallas as pl
from jax.experimental.pallas import tpu as pltpu

_WINDOW = 4
_K = 8
_QBLK = 256


def _fold_kernel(wv_ref, wm2_ref, wo_ref, bv_ref, bm2_ref, bo_ref,
                 wvpT_ref, wlT_ref, bvp_ref, ball_ref):
    D = wv_ref.shape[0]
    wo = wo_ref[...]
    wol = wo[:, :D]
    wog = wo[:, D:]
    wv = wv_ref[...]
    wm2 = wm2_ref[...]
    wvpT_ref[...] = lax.dot_general(
        wv, wog, (((0,), (1,)), ((), ())),
        preferred_element_type=jnp.float32).astype(jnp.bfloat16)
    wlT_ref[...] = lax.dot_general(
        wm2, wol, (((0,), (1,)), ((), ())),
        preferred_element_type=jnp.float32).astype(jnp.bfloat16)
    bvp_ref[...] = lax.dot_general(
        bv_ref[...], wog, (((1,), (1,)), ((), ())),
        preferred_element_type=jnp.float32)
    ball_ref[...] = bo_ref[...] + lax.dot_general(
        bm2_ref[...], wol, (((1,), (1,)), ((), ())),
        preferred_element_type=jnp.float32)


def _proj_kernel(nblk, x_ref, wk_ref, wvpT_ref, wm1_ref, wlT_ref,
                 bk_ref, bvp_ref, bm1_ref, ball_ref,
                 k_ref, vp_ref, loc_ref, tail_ref):
    i = pl.program_id(0)
    qi = lax.rem(i, nblk)
    D = x_ref.shape[1]
    QB = x_ref.shape[0]
    x = x_ref[...]
    k_ref[...] = lax.dot_general(
        x, wk_ref[...], (((1,), (1,)), ((), ())),
        preferred_element_type=jnp.float32) + bk_ref[...]
    xb = x.astype(jnp.bfloat16)
    vp = lax.dot_general(
        xb, wvpT_ref[...], (((1,), (0,)), ((), ())),
        preferred_element_type=jnp.float32) + bvp_ref[...]
    vp_ref[...] = vp.astype(jnp.bfloat16)
    wm1 = wm1_ref[...]
    a = lax.dot_general(
        xb, wm1[:, :D].astype(jnp.bfloat16), (((1,), (1,)), ((), ())),
        preferred_element_type=jnp.float32) + bm1_ref[...]
    bn = lax.dot_general(
        xb, wm1[:, D:].astype(jnp.bfloat16), (((1,), (1,)), ((), ())),
        preferred_element_type=jnp.float32)
    prev = jnp.where(qi == 0, 0.0, tail_ref[...])
    acc = jnp.zeros_like(a)
    for w in range(1, _WINDOW + 1):
        sh = jnp.concatenate([prev[8 - w:, :], bn[:QB - w, :]], axis=0)
        z = a + sh
        acc = acc + z * jax.nn.sigmoid(z)
    tail_ref[...] = bn[QB - 8:, :]
    pre = (acc * (1.0 / _WINDOW)).astype(jnp.bfloat16)
    loc_ref[...] = lax.dot_general(
        pre, wlT_ref[...], (((1,), (0,)), ((), ())),
        preferred_element_type=jnp.float32) + ball_ref[...]


def _attn_kernel(x_ref, wq_ref, bq_ref, k_ref, vp_ref, loc_ref, o_ref):
    qi = pl.program_id(1)
    QB = x_ref.shape[1]
    N = k_ref.shape[1]
    D = x_ref.shape[2]
    scale = 1.0 / math.sqrt(D)
    q = lax.dot_general(
        x_ref[0], wq_ref[...], (((1,), (1,)), ((), ())),
        preferred_element_type=jnp.float32) + bq_ref[...]
    s = lax.dot_general(
        q, k_ref[0], (((1,), (1,)), ((), ())),
        preferred_element_type=jnp.float32) * scale
    row = qi * QB + lax.broadcasted_iota(jnp.int32, (QB, N), 0)
    col = lax.broadcasted_iota(jnp.int32, (QB, N), 1)
    neg = jnp.float32(-jnp.inf)
    s = jnp.where(col <= row, s, neg)
    work = s
    m0 = None
    for j in range(_K):
        m = jnp.max(work, axis=1, keepdims=True)
        if j == 0:
            m0 = m
        am = jnp.min(jnp.where(work == m, col, N), axis=1, keepdims=True)
        work = jnp.where(col == am, neg, work)
    mask = work != s
    p = jnp.where(mask, jnp.exp(s - m0), 0.0)
    z = jnp.sum(p, axis=1, keepdims=True)
    g = lax.dot_general(
        p.astype(jnp.bfloat16), vp_ref[0], (((1,), (0,)), ((), ())),
        preferred_element_type=jnp.float32)
    o_ref[0] = g / z + loc_ref[0]


def kernel(mu, Wq, bq, Wk, bk, Wv, bv, Wm1, bm1, Wm2, bm2, Wo, bo):
    B, N, D = mu.shape
    f32 = jnp.float32
    bf16 = jnp.bfloat16

    wvpT, wlT, bvp, ball = pl.pallas_call(
        _fold_kernel,
        out_shape=[
            jax.ShapeDtypeStruct((D, D), bf16),
            jax.ShapeDtypeStruct((D, D), bf16),
            jax.ShapeDtypeStruct((1, D), f32),
            jax.ShapeDtypeStruct((1, D), f32),
        ],
    )(Wv, Wm2, Wo, bv[None, :], bm2[None, :], bo[None, :])

    mu_flat = mu.reshape(B * N, D)
    nblk = N // _QBLK
    import functools
    kk, vp, loc = pl.pallas_call(
        functools.partial(_proj_kernel, nblk),
        grid=(B * nblk,),
        in_specs=[
            pl.BlockSpec((_QBLK, D), lambda i: (i, 0)),
            pl.BlockSpec((D, D), lambda i: (0, 0)),
            pl.BlockSpec((D, D), lambda i: (0, 0)),
            pl.BlockSpec((D, 2 * D), lambda i: (0, 0)),
            pl.BlockSpec((D, D), lambda i: (0, 0)),
            pl.BlockSpec((1, D), lambda i: (0, 0)),
            pl.BlockSpec((1, D), lambda i: (0, 0)),
            pl.BlockSpec((1, D), lambda i: (0, 0)),
            pl.BlockSpec((1, D), lambda i: (0, 0)),
        ],
        out_specs=[
            pl.BlockSpec((_QBLK, D), lambda i: (i, 0)),
            pl.BlockSpec((_QBLK, D), lambda i: (i, 0)),
            pl.BlockSpec((_QBLK, D), lambda i: (i, 0)),
        ],
        out_shape=[
            jax.ShapeDtypeStruct((B * N, D), f32),
            jax.ShapeDtypeStruct((B * N, D), bf16),
            jax.ShapeDtypeStruct((B * N, D), f32),
        ],
        scratch_shapes=[pltpu.VMEM((8, D), f32)],
    )(mu_flat, Wk, wvpT, Wm1, wlT,
      bk[None, :], bvp, bm1[None, :], ball)

    kk = kk.reshape(B, N, D)
    vp = vp.reshape(B, N, D)
    loc = loc.reshape(B, N, D)

    out = pl.pallas_call(
        _attn_kernel,
        grid=(B, nblk),
        in_specs=[
            pl.BlockSpec((1, _QBLK, D), lambda b, i: (b, i, 0)),
            pl.BlockSpec((D, D), lambda b, i: (0, 0)),
            pl.BlockSpec((1, D), lambda b, i: (0, 0)),
            pl.BlockSpec((1, N, D), lambda b, i: (b, 0, 0)),
            pl.BlockSpec((1, N, D), lambda b, i: (b, 0, 0)),
            pl.BlockSpec((1, _QBLK, D), lambda b, i: (b, i, 0)),
        ],
        out_specs=pl.BlockSpec((1, _QBLK, D), lambda b, i: (b, i, 0)),
        out_shape=jax.ShapeDtypeStruct((B, N, D), f32),
    )(mu, Wq, bq[None, :], kk, vp, loc)
    return out
